# baseline (speedup 1.0000x reference)
import jax
import jax.numpy as jnp
from jax.experimental import pallas as pl
from jax.experimental.pallas import tpu as pltpu

N = 2048
H = 256
B = 1024
C = 4
R = N // C
_F32 = jnp.float32


def _elu(x):
    return jnp.where(x > 0, x, jnp.exp(jnp.minimum(x, 0.0)) - 1.0)


def _dot(a, b):
    return jnp.dot(a, b, preferred_element_type=_F32)


def _dotT(a, b):
    return jax.lax.dot_general(a, b, (((0,), (0,)), ((), ())),
                               preferred_element_type=_F32)


def _mega_body(xk_ab, xp_ab, xk_v, xp_v, ai, vi,
               wk_ab, bk_ab, wp_ab, bp_ab, wk_v, bk_v, wp_v, bp_v,
               wsk, wsp, b2, wt_ab, bt_ab, wt_v, bt_v,
               wg1, bg1, wg2, bg2, wgt, wgb, bg, wpr, bpr,
               out_ref,
               node_ab, node_v, that_ab, that_v,
               s1_ab, s1_v, gn_ab, gn_v, gt_ab, gt_v, ga, gv):
    i = pl.program_id(0)

    def pass_a(xk, xp, wk, bk, wp, bp, wt, bt, idx_ref,
               node_s, that_s, s1_s, gn_s, gt_s, c):
        rows = pl.ds(c * R, R)
        ak = _elu(_dot(xk[:], wk[:]) + bk[:])
        ap = _elu(_dot(xp[:], wp[:]) + bp[:])
        node_c = _dot(ak, wsk[:]) + _dot(ap, wsp[:]) + b2[:]
        node_s[rows, :] = node_c
        ne = _elu(node_c)
        tr = jnp.tanh(_dot(ne, wt[:]) + bt[:])
        inv = jax.lax.rsqrt(jnp.sum(tr * tr, axis=1, keepdims=True))
        that_c = tr * inv
        that_s[rows, :] = that_c
        y1 = _dot(ne, wg1[:])
        s1_c = _dotT(that_c, y1)
        iota = jax.lax.broadcasted_iota(jnp.int32, (B, R), 1) + c * R
        onehot = (iota == idx_ref[:]).astype(_F32)
        gn_c = _dot(onehot, node_c)
        gt_c = _dot(onehot, that_c)
        if c == 0:
            s1_s[:] = s1_c
            gn_s[:] = gn_c
            gt_s[:] = gt_c
        else:
            s1_s[:] = s1_s[:] + s1_c
            gn_s[:] = gn_s[:] + gn_c
            gt_s[:] = gt_s[:] + gt_c

    for c in range(C):
        @pl.when(i == c)
        def _(c=c):
            pass_a(xk_ab, xp_ab, wk_ab, bk_ab, wp_ab, bp_ab,
                   wt_ab, bt_ab, ai, node_ab, that_ab, s1_ab,
                   gn_ab, gt_ab, c)
            pass_a(xk_v, xp_v, wk_v, bk_v, wp_v, bp_v,
                   wt_v, bt_v, vi, node_v, that_v, s1_v,
                   gn_v, gt_v, c)

    def pass_b(node_s, that_s, s1_s, gn_s, gt_s, g_ref):
        that = that_s[:]
        res1 = node_s[:] + _dot(that, s1_s[:]) + bg1[:]
        y2 = _dot(_elu(res1), wg2[:])
        s2 = _dotT(that, y2)
        g_ref[:] = (gn_s[:] + _dot(gt_s[:], s1_s[:] + s2)
                    + bg1[:] + bg2[:])

    @pl.when(i == C)
    def _():
        pass_b(node_ab, that_ab, s1_ab, gn_ab, gt_ab, ga)

    @pl.when(i == C + 1)
    def _():
        pass_b(node_v, that_v, s1_v, gn_v, gt_v, gv)
        h = _elu(_dot(_elu(ga[:]), wgt[:]) + _dot(_elu(gv[:]), wgb[:])
                 + bg[:])
        out_ref[:] = _dot(h, wpr[:]) + bpr[:]


def kernel(antibody_graph_node_kmer_ft, antibody_graph_node_pssm_ft,
           virus_graph_node_kmer_ft, virus_graph_node_pssm_ft,
           antibody_idx, virus_idx, W_ab_k, b_ab_k, W_ab_p, b_ab_p,
           W_v_k, b_v_k, W_v_p, b_v_p, conv_w, conv_b, W_share, b_share,
           W_g1, b_g1, W_g2, b_g2, W_ab_t, b_ab_t, W_v_t, b_v_t,
           W_glob, b_glob, W_pred, b_pred):
    ws2 = (conv_w[None, :, None] * W_share[:, None, :]).reshape(2 * H, H)
    wsk, wsp = ws2[:H], ws2[H:]
    b2 = (b_share + conv_b * jnp.sum(W_share, axis=0)).reshape(1, H)

    row = lambda b: b.reshape(1, -1)
    ai = antibody_idx.astype(jnp.int32).reshape(B, 1)
    vi = virus_idx.astype(jnp.int32).reshape(B, 1)

    kp = antibody_graph_node_kmer_ft.shape[1]
    pa = antibody_graph_node_pssm_ft.shape[1]
    pv = virus_graph_node_pssm_ft.shape[1]

    def tile(i):
        return (jnp.minimum(i, C - 1), 0)

    def full(a):
        return pl.BlockSpec(a.shape, lambda i: (0,) * a.ndim)

    weights = [W_ab_k, row(b_ab_k), W_ab_p, row(b_ab_p),
               W_v_k, row(b_v_k), W_v_p, row(b_v_p),
               wsk, wsp, b2, W_ab_t, row(b_ab_t), W_v_t, row(b_v_t),
               W_g1, row(b_g1), W_g2, row(b_g2),
               W_glob[:H], W_glob[H:], row(b_glob), W_pred, row(b_pred)]

    out = pl.pallas_call(
        _mega_body,
        grid=(C + 2,),
        in_specs=[
            pl.BlockSpec((R, kp), tile),
            pl.BlockSpec((R, pa), tile),
            pl.BlockSpec((R, kp), tile),
            pl.BlockSpec((R, pv), tile),
            full(ai),
            full(vi),
        ] + [full(w) for w in weights],
        out_specs=pl.BlockSpec((B, 1), lambda i: (0, 0)),
        out_shape=jax.ShapeDtypeStruct((B, 1), _F32),
        scratch_shapes=[
            pltpu.VMEM((N, H), _F32),
            pltpu.VMEM((N, H), _F32),
            pltpu.VMEM((N, H), _F32),
            pltpu.VMEM((N, H), _F32),
            pltpu.VMEM((H, H), _F32),
            pltpu.VMEM((H, H), _F32),
            pltpu.VMEM((B, H), _F32),
            pltpu.VMEM((B, H), _F32),
            pltpu.VMEM((B, H), _F32),
            pltpu.VMEM((B, H), _F32),
            pltpu.VMEM((B, H), _F32),
            pltpu.VMEM((B, H), _F32),
        ],
    )(antibody_graph_node_kmer_ft, antibody_graph_node_pssm_ft,
      virus_graph_node_kmer_ft, virus_graph_node_pssm_ft, ai, vi,
      *weights)
    return out


# speedup vs baseline: 1.0245x; 1.0245x over previous
import jax
import jax.numpy as jnp
from jax.experimental import pallas as pl
from jax.experimental.pallas import tpu as pltpu

N = 2048
H = 256
B = 1024
C = 2
R = N // C
_F32 = jnp.float32


def _elu(x):
    return jnp.where(x > 0, x, jnp.exp(jnp.minimum(x, 0.0)) - 1.0)


def _dot(a, b):
    return jnp.dot(a, b, preferred_element_type=_F32)


def _dotT(a, b):
    return jax.lax.dot_general(a, b, (((0,), (0,)), ((), ())),
                               preferred_element_type=_F32)


def _mega_body(xk_ab, xp_ab, xk_v, xp_v, ai, vi,
               wk_ab, bk_ab, wp_ab, bp_ab, wk_v, bk_v, wp_v, bp_v,
               wsk, wsp, b2, wt_ab, bt_ab, wt_v, bt_v,
               wg1, bg1, wg2, bg2, wgt, wgb, bg, wpr, bpr,
               out_ref, vk_ab, vp_ab, vk_v, vp_v, node, ga, gv,
               semk, semp):

    def start(src, dst, sem, c):
        cp = pltpu.make_async_copy(
            src.at[pl.ds(c * R, R), :], dst.at[pl.ds(c * R, R), :], sem)
        cp.start()
        return cp

    copies = []
    for c in range(C):
        copies.append((start(xk_ab, vk_ab, semk.at[c], c),
                       start(xp_ab, vp_ab, semp.at[c], c)))
    for c in range(C):
        copies.append((start(xk_v, vk_v, semk.at[C + c], c),
                       start(xp_v, vp_v, semp.at[C + c], c)))

    def stage1(vk, vp, wk, bk, wp, bp, c):
        rows = pl.ds(c * R, R)
        ak = _elu(_dot(vk[rows, :], wk[:]) + bk[:])
        ap = _elu(_dot(vp[rows, :], wp[:]) + bp[:])
        node[rows, :] = _dot(ak, wsk[:]) + _dot(ap, wsp[:]) + b2[:]

    def stage2(idx_ref, wt, bt, g_ref):
        nd = node[:]
        res = nd
        ne = _elu(nd)
        trans = jnp.tanh(_dot(ne, wt[:]) + bt[:])
        inv = jax.lax.rsqrt(jnp.sum(trans * trans, axis=1, keepdims=True))
        that = trans * inv
        y = _dot(ne, wg1[:])
        res = res + _dot(that, _dotT(that, y)) + bg1[:]
        ne = _elu(res)
        y = _dot(ne, wg2[:])
        res = res + _dot(that, _dotT(that, y)) + bg2[:]
        iota = jax.lax.broadcasted_iota(jnp.int32, (B, N), 1)
        onehot = (iota == idx_ref[:]).astype(_F32)
        g_ref[:] = _dot(onehot, res)

    for c in range(C):
        copies[c][0].wait()
        copies[c][1].wait()
        stage1(vk_ab, vp_ab, wk_ab, bk_ab, wp_ab, bp_ab, c)
    stage2(ai, wt_ab, bt_ab, ga)

    for c in range(C):
        copies[C + c][0].wait()
        copies[C + c][1].wait()
        stage1(vk_v, vp_v, wk_v, bk_v, wp_v, bp_v, c)
    stage2(vi, wt_v, bt_v, gv)

    h = _elu(_dot(_elu(ga[:]), wgt[:]) + _dot(_elu(gv[:]), wgb[:]) + bg[:])
    out_ref[:] = _dot(h, wpr[:]) + bpr[:]


def kernel(antibody_graph_node_kmer_ft, antibody_graph_node_pssm_ft,
           virus_graph_node_kmer_ft, virus_graph_node_pssm_ft,
           antibody_idx, virus_idx, W_ab_k, b_ab_k, W_ab_p, b_ab_p,
           W_v_k, b_v_k, W_v_p, b_v_p, conv_w, conv_b, W_share, b_share,
           W_g1, b_g1, W_g2, b_g2, W_ab_t, b_ab_t, W_v_t, b_v_t,
           W_glob, b_glob, W_pred, b_pred):
    ws2 = (conv_w[None, :, None] * W_share[:, None, :]).reshape(2 * H, H)
    wsk, wsp = ws2[:H], ws2[H:]
    b2 = (b_share + conv_b * jnp.sum(W_share, axis=0)).reshape(1, H)

    row = lambda b: b.reshape(1, -1)
    ai = antibody_idx.astype(jnp.int32).reshape(B, 1)
    vi = virus_idx.astype(jnp.int32).reshape(B, 1)

    kp = antibody_graph_node_kmer_ft.shape[1]
    pa = antibody_graph_node_pssm_ft.shape[1]
    pv = virus_graph_node_pssm_ft.shape[1]

    hbm = pl.BlockSpec(memory_space=pltpu.MemorySpace.HBM)
    weights = [W_ab_k, row(b_ab_k), W_ab_p, row(b_ab_p),
               W_v_k, row(b_v_k), W_v_p, row(b_v_p),
               wsk, wsp, b2, W_ab_t, row(b_ab_t), W_v_t, row(b_v_t),
               W_g1, row(b_g1), W_g2, row(b_g2),
               W_glob[:H], W_glob[H:], row(b_glob), W_pred, row(b_pred)]

    out = pl.pallas_call(
        _mega_body,
        in_specs=[hbm, hbm, hbm, hbm]
        + [pl.BlockSpec(memory_space=pltpu.MemorySpace.VMEM)] * (2 + len(weights)),
        out_shape=jax.ShapeDtypeStruct((B, 1), _F32),
        scratch_shapes=[
            pltpu.VMEM((N, kp), _F32),
            pltpu.VMEM((N, pa), _F32),
            pltpu.VMEM((N, kp), _F32),
            pltpu.VMEM((N, pv), _F32),
            pltpu.VMEM((N, H), _F32),
            pltpu.VMEM((B, H), _F32),
            pltpu.VMEM((B, H), _F32),
            pltpu.SemaphoreType.DMA((2 * C,)),
            pltpu.SemaphoreType.DMA((2 * C,)),
        ],
    )(antibody_graph_node_kmer_ft, antibody_graph_node_pssm_ft,
      virus_graph_node_kmer_ft, virus_graph_node_pssm_ft, ai, vi,
      *weights)
    return out
